# Initial kernel scaffold; baseline (speedup 1.0000x reference)
import jax
import jax.numpy as jnp
from jax.experimental import pallas as pl


def kernel(frame_type_ids, frame_tokens):
    raise NotImplementedError("write your pallas kernel here")


# speedup vs baseline: 1.0802x; 1.0802x over previous
import functools

import jax
import jax.numpy as jnp
from jax import lax
from jax.experimental import pallas as pl
from jax.experimental.pallas import tpu as pltpu
from jax.experimental.pallas import tpu_sc as plsc

NC = 2
NS = 16
L = 16


def kernel(frame_type_ids, frame_tokens):
    B = frame_type_ids.shape[0]
    T, TOK, H = frame_tokens.shape
    D = TOK * H
    NW = NC * NS
    b_per_w = B // NW
    groups = b_per_w // L

    table = frame_tokens.reshape(T, D)
    ids = frame_type_ids.astype(jnp.int32)

    mesh = plsc.VectorSubcoreMesh(core_axis_name="c", subcore_axis_name="s")

    @functools.partial(
        pl.kernel,
        out_type=jax.ShapeDtypeStruct((B, D), jnp.float32),
        mesh=mesh,
        scratch_types=[
            pltpu.VMEM((T, D), jnp.float32),
            pltpu.VMEM((b_per_w,), jnp.int32),
            pltpu.SemaphoreType.DMA,
        ],
    )
    def run(ids_hbm, table_hbm, out_hbm, table_v, idx_v, sem):
        wid = lax.axis_index("s") * NC + lax.axis_index("c")
        base = wid * b_per_w
        pltpu.sync_copy(table_hbm, table_v)
        pltpu.sync_copy(ids_hbm.at[pl.ds(base, b_per_w)], idx_v)

        def group_body(g, _):
            vec = idx_v[pl.ds(g * L, L)]
            for e in range(L):
                id_e = vec[e]
                b = base + g * L + e
                pltpu.async_copy(table_v.at[id_e], out_hbm.at[b], sem)
            return 0

        lax.fori_loop(0, groups, group_body, 0)

        def drain_body(i, _):
            pltpu.make_async_copy(table_v.at[0], out_hbm.at[base], sem).wait()
            return 0

        lax.fori_loop(0, b_per_w, drain_body, 0)

    out = run(ids, table)
    return out.reshape(B, TOK, H)


# speedup vs baseline: 4.9323x; 4.5661x over previous
import functools

import jax
import jax.numpy as jnp
from jax import lax
from jax.experimental import pallas as pl
from jax.experimental.pallas import tpu as pltpu
from jax.experimental.pallas import tpu_sc as plsc

NC = 2
NS = 16
L = 16


def kernel(frame_type_ids, frame_tokens):
    B = frame_type_ids.shape[0]
    T, TOK, H = frame_tokens.shape
    D = TOK * H
    NW = NC * NS
    b_per_w = B // NW
    groups = b_per_w // L

    ids = frame_type_ids.astype(jnp.int32)

    mesh = plsc.VectorSubcoreMesh(core_axis_name="c", subcore_axis_name="s")

    @functools.partial(
        pl.kernel,
        out_type=jax.ShapeDtypeStruct((B, TOK, H), jnp.float32),
        mesh=mesh,
        scratch_types=[
            pltpu.VMEM((T, TOK, H), jnp.float32),
            pltpu.VMEM((b_per_w,), jnp.int32),
            pltpu.SemaphoreType.DMA,
        ],
    )
    def run(ids_hbm, table_hbm, out_hbm, table_v, idx_v, sem):
        wid = lax.axis_index("s") * NC + lax.axis_index("c")
        base = wid * b_per_w
        pltpu.sync_copy(table_hbm, table_v)
        pltpu.sync_copy(ids_hbm.at[pl.ds(base, b_per_w)], idx_v)

        def group_body(g, _):
            vec = idx_v[pl.ds(g * L, L)]
            for e in range(L):
                id_e = vec[e]
                b = base + g * L + e
                pltpu.async_copy(table_v.at[id_e], out_hbm.at[b], sem)
            return 0

        lax.fori_loop(0, groups, group_body, 0)

        def drain_body(i, _):
            pltpu.make_async_copy(table_v.at[0], out_hbm.at[base], sem).wait()
            return 0

        lax.fori_loop(0, b_per_w, drain_body, 0)

    return run(ids, frame_tokens)


# speedup vs baseline: 4.9433x; 1.0022x over previous
import functools

import jax
import jax.numpy as jnp
from jax import lax
from jax.experimental import pallas as pl
from jax.experimental.pallas import tpu as pltpu
from jax.experimental.pallas import tpu_sc as plsc

NC = 2
NS = 16
L = 16


def kernel(frame_type_ids, frame_tokens):
    B = frame_type_ids.shape[0]
    T, TOK, H = frame_tokens.shape
    D = TOK * H
    NW = NC * NS
    b_per_w = B // NW
    groups = b_per_w // L

    ids = frame_type_ids.astype(jnp.int32)

    mesh = plsc.VectorSubcoreMesh(core_axis_name="c", subcore_axis_name="s")

    @functools.partial(
        pl.kernel,
        out_type=jax.ShapeDtypeStruct((B, TOK, H), jnp.float32),
        mesh=mesh,
        scratch_types=[
            pltpu.VMEM((T, TOK, H), jnp.float32),
            pltpu.VMEM((b_per_w,), jnp.int32),
            pltpu.SemaphoreType.DMA,
        ],
    )
    def run(ids_hbm, table_hbm, out_hbm, table_v, idx_v, sem):
        wid = lax.axis_index("s") * NC + lax.axis_index("c")
        base = wid * b_per_w
        rot = lax.rem(wid, T)
        for k in range(T):
            r = lax.rem(rot + k, T)
            pltpu.async_copy(table_hbm.at[r], table_v.at[r], sem)
        pltpu.async_copy(ids_hbm.at[pl.ds(base, b_per_w)], idx_v, sem)
        for k in range(T):
            pltpu.make_async_copy(table_hbm.at[0], table_v.at[0], sem).wait()
        pltpu.make_async_copy(ids_hbm.at[pl.ds(0, b_per_w)], idx_v, sem).wait()

        def group_body(g, _):
            vec = idx_v[pl.ds(g * L, L)]
            for e in range(L):
                id_e = vec[e]
                b = base + g * L + e
                pltpu.async_copy(table_v.at[id_e], out_hbm.at[b], sem)
            return 0

        lax.fori_loop(0, groups, group_body, 0)

        def drain_body(i, _):
            pltpu.make_async_copy(table_v.at[0], out_hbm.at[base], sem).wait()
            return 0

        lax.fori_loop(0, b_per_w, drain_body, 0)

    return run(ids, frame_tokens)
